# Initial kernel scaffold; baseline (speedup 1.0000x reference)
import jax
import jax.numpy as jnp
from jax.experimental import pallas as pl


def kernel(pos, pcl, pre_oriented_normal, params):
    raise NotImplementedError("write your pallas kernel here")


# speedup vs baseline: 18.6549x; 18.6549x over previous
import functools
import jax
import jax.numpy as jnp
from jax import lax
from jax.experimental import pallas as pl
from jax.experimental.pallas import tpu as pltpu

F32 = jnp.float32
I32 = jnp.int32
NEG = -jnp.inf


def _ceil128(n):
    return (n + 127) // 128 * 128


def _gather_cols(tab_T, ik_row, Q):
    C, R = tab_T.shape
    W = ik_row.shape[1]
    Rp = _ceil128(R)
    if Rp != R:
        tab_T = jnp.concatenate(
            [tab_T, jnp.zeros((C, Rp - R), dtype=F32)], axis=1)
    idxb = jnp.broadcast_to(ik_row, (C, W))
    loc = idxb % 128
    ch = idxb // 128
    chunks = [tab_T[:, j * 128:(j + 1) * 128] for j in range(Rp // 128)]
    tiles = []
    for i in range(W // 128):
        loc_i = loc[:, i * 128:(i + 1) * 128]
        ch_i = ch[:, i * 128:(i + 1) * 128]
        acc = jnp.zeros((C, 128), dtype=F32)
        for j, chunk in enumerate(chunks):
            g = jnp.take_along_axis(chunk, loc_i, axis=1,
                                    mode="promise_in_bounds")
            acc = jnp.where(ch_i == j, g, acc)
        tiles.append(acc)
    return jnp.concatenate(tiles, axis=1)[:, :Q]


def _topk_min_idx(d_T, K, d_scr, idx_scr):
    R, Q = d_T.shape
    S, W = d_scr.shape
    d_scr[:, :] = jnp.full((S, W), jnp.inf, dtype=F32)
    d_scr[:R, :Q] = d_T
    iot = lax.broadcasted_iota(I32, (S, W), 0)

    def body(k, carry):
        d = d_scr[:, :]
        m = jnp.min(d, axis=0, keepdims=True)
        cand = jnp.where(d == m, iot, jnp.int32(2 ** 30))
        idxv = jnp.min(cand, axis=0, keepdims=True)
        idx_scr[pl.ds(k, 1), :] = idxv
        d_scr[:, :] = jnp.where(iot == idxv, jnp.inf, d)
        return carry

    lax.fori_loop(0, K, body, 0)


def _edge_conv(x_T, W, K, k_off, idx_scr, N):
    d = x_T.shape[0]
    Wt, Wb = W[:d], W[d:]
    A = lax.dot_general(Wt - Wb, x_T, (((0,), (0,)), ((), ())),
                        preferred_element_type=F32)
    Bv = lax.dot_general(Wb, x_T, (((0,), (0,)), ((), ())),
                         preferred_element_type=F32)

    def body(k, acc):
        ik = idx_scr[pl.ds(k_off + k, 1), :]
        g = _gather_cols(Bv, ik, N)
        return jnp.maximum(acc, jax.nn.relu(A + g))

    acc0 = jnp.full((24, N), NEG, dtype=F32)
    return lax.fori_loop(0, K, body, acc0)


def _local_feat(pos_T, Ws, K, k_off, idx_scr):
    x_T = pos_T
    N = pos_T.shape[1]
    for W in Ws:
        e = _edge_conv(x_T, W, K, k_off, idx_scr, N)
        x_T = jnp.concatenate([x_T, e], axis=0)
    return x_T


def _gather_max(y_T, K, k_off, idx_scr, Q):
    C = y_T.shape[0]

    def body(k, acc):
        ik = idx_scr[pl.ds(k_off + k, 1), :]
        g = _gather_cols(y_T, ik, Q)
        return jnp.maximum(acc, g)

    acc0 = jnp.full((C, Q), NEG, dtype=F32)
    return lax.fori_loop(0, K, body, acc0)


def _mm_T(W, x):
    return lax.dot_general(W, x, (((0,), (0,)), ((), ())),
                           preferred_element_type=F32)


def _dist_T(ref_rm, ref_sq, query_T, query_sq, R, Q):
    mm = lax.dot_general(ref_rm[:R], query_T[:, :Q],
                         (((1,), (0,)), ((), ())),
                         preferred_element_type=F32)
    return (ref_sq[:R] + query_sq[:, :Q]) - 2.0 * mm


def _qstn(pos_rm, Ws):
    h = jax.nn.relu(jnp.dot(pos_rm, Ws[0], preferred_element_type=F32))
    h = jax.nn.relu(jnp.dot(h, Ws[1], preferred_element_type=F32))
    h = jax.nn.relu(jnp.dot(h, Ws[2], preferred_element_type=F32))
    h = jnp.max(h, axis=0, keepdims=True)
    h = jax.nn.relu(jnp.dot(h, Ws[3], preferred_element_type=F32))
    h = jax.nn.relu(jnp.dot(h, Ws[4], preferred_element_type=F32))
    q = jnp.dot(h, Ws[5], preferred_element_type=F32)
    lane4 = lax.broadcasted_iota(I32, (1, 4), 1)
    q = q + jnp.where(lane4 == 0, 1.0, 0.0).astype(F32)
    q = q / (jnp.sqrt(jnp.sum(q * q, axis=1, keepdims=True)) + 1e-8)
    w, x, y, z = (q[:, 0:1], q[:, 1:2], q[:, 2:3], q[:, 3:4])
    r0 = jnp.concatenate([1 - 2 * (y * y + z * z), 2 * (x * y - w * z),
                          2 * (x * z + w * y)], axis=1)
    r1 = jnp.concatenate([2 * (x * y + w * z), 1 - 2 * (x * x + z * z),
                          2 * (y * z - w * x)], axis=1)
    r2 = jnp.concatenate([2 * (x * z - w * y), 2 * (y * z + w * x),
                          1 - 2 * (x * x + y * y)], axis=1)
    return jnp.concatenate([r0, r1, r2], axis=0)


def _hier(pooled_T, W, Wg, M, x_last=None, Wl=None):
    h = jax.nn.relu(_mm_T(W, pooled_T))
    if x_last is not None:
        h = h + jax.nn.relu(_mm_T(Wl, x_last))
    g = jnp.max(jax.nn.relu(_mm_T(Wg, h)), axis=1, keepdims=True)
    return h, g


_N = 700
_M0, _M1, _M2, _M3 = 466, 310, 206, 103
_KL1, _KL2, _KH1, _KH2 = 16, 32, 16, 32
_SW = 768


def _main_kernel(pos_ref, pre_ref, gg_ref,
                 stn0, stn1, stn2, stn3, stn4, stn5,
                 lf1_0, lf1_1, lf1_2, lf1_3,
                 lf2_0, lf2_1, lf2_2, lf2_3,
                 alpha_ref, c1_ref, c2_ref,
                 s1_0, s1_1, s2_0, s2_1, s2_2,
                 s3_0, s3_1, s3_2, s4_0, s4_1, s4_2,
                 c3_ref, c4_ref, cg_ref, mg0, mg1,
                 k1_ref, k2_ref, kw_ref, n_ref,
                 sg0, sg1, sg2,
                 out_ref, d_scr, idx_scr):
    pos = pos_ref[0]
    rot = _qstn(pos, [stn0[...], stn1[...], stn2[...],
                      stn3[...], stn4[...], stn5[...]])
    pos_rm = jnp.dot(pos, rot, preferred_element_type=F32)
    pos_T = pos_rm.T
    psq_rm = jnp.sum(pos_rm * pos_rm, axis=1, keepdims=True)
    psq_T = jnp.sum(pos_T * pos_T, axis=0, keepdims=True)

    d_T = _dist_T(pos_rm, psq_rm, pos_T, psq_T, _N, _N)
    _topk_min_idx(d_T, _KL2 + 1, d_scr, idx_scr)

    y1 = _local_feat(pos_T, [lf1_0[...], lf1_1[...], lf1_2[...], lf1_3[...]],
                     _KL1, 1, idx_scr)
    y2 = _local_feat(pos_T, [lf2_0[...], lf2_1[...], lf2_2[...], lf2_3[...]],
                     _KL2, 1, idx_scr)
    a = jax.nn.sigmoid(alpha_ref[...])
    y = a * y1 + (1.0 - a) * y2
    y = jax.nn.relu(_mm_T(c1_ref[...], y))
    y = jax.nn.relu(_mm_T(c2_ref[...], y))

    d_T = _dist_T(pos_rm, psq_rm, pos_T, psq_T, _N, _M0)
    _topk_min_idx(d_T, _KH1 + 1, d_scr, idx_scr)
    pooled = _gather_max(y, _KH1, 1, idx_scr, _M0)
    y, g1 = _hier(pooled, s1_0[...], s1_1[...], _M0)

    d_T = _dist_T(pos_rm, psq_rm, pos_T, psq_T, _M0, _M1)
    _topk_min_idx(d_T, _KH1 + 1, d_scr, idx_scr)
    pooled = _gather_max(y, _KH1, 1, idx_scr, _M1)
    y, g2 = _hier(pooled, s2_0[...], s2_1[...], _M1, g1, s2_2[...])

    d_T = _dist_T(pos_rm, psq_rm, pos_T, psq_T, _M1, _M2)
    _topk_min_idx(d_T, _KH2 + 1, d_scr, idx_scr)
    pooled = _gather_max(y, _KH2, 1, idx_scr, _M2)
    y, g3 = _hier(pooled, s3_0[...], s3_1[...], _M2, g2, s3_2[...])

    d_T = _dist_T(pos_rm, psq_rm, pos_T, psq_T, _M2, _M2)
    _topk_min_idx(d_T, _KH2 + 1, d_scr, idx_scr)
    pooled = _gather_max(y, _KH2, 1, idx_scr, _M2)
    y, g4 = _hier(pooled, s4_0[...], s4_1[...], _M2, g3, s4_2[...])

    y = jax.nn.relu(_mm_T(c3_ref[...], y)) + y
    y = jax.nn.relu(_mm_T(c4_ref[...], y))
    yg = jax.nn.relu(_mm_T(cg_ref[...], y[:, :_M3])) + y[:, :_M3]
    yg = jnp.max(yg, axis=1, keepdims=True)
    pg = jnp.concatenate([g1, g2, g3, g4, yg], axis=0)
    pg = jax.nn.relu(_mm_T(mg0[...], pg))
    pg = jax.nn.relu(_mm_T(mg1[...], pg))

    h = jax.nn.relu(_mm_T(k1_ref[...], y))
    h = jax.nn.relu(_mm_T(k2_ref[...], h))
    logit = _mm_T(kw_ref[...], h)
    logit = logit - jnp.max(logit, axis=1, keepdims=True)
    wexp = jnp.exp(logit)
    wsm = wexp / jnp.sum(wexp, axis=1, keepdims=True)
    f = jnp.sum(y * wsm, axis=1, keepdims=True)
    n = _mm_T(n_ref[...], f)
    comb = jnp.concatenate([f, pg, gg_ref[0]], axis=0)
    s = jax.nn.relu(_mm_T(sg0[...], comb))
    s = jax.nn.relu(_mm_T(sg1[...], s))
    s = _mm_T(sg2[...], s)
    nu = n / (jnp.sqrt(jnp.sum(n * n, axis=0, keepdims=True)) + 1e-8)
    flip = jnp.sign(jnp.sum(nu * pre_ref[0], axis=0, keepdims=True) + 1e-6)
    out_ref[0] = nu * jnp.tanh(s) * flip


_NG = 1024
_MG0, _MG1 = 512, 256
_KG = 8


def _g_kernel(pcl_ref,
              lfg_0, lfg_1, lfg_2, lfg_3,
              gc1_ref, gc2_ref,
              gs1_0, gs1_1, gs2_0, gs2_1, gs2_2, gs3_0, gs3_1, gs3_2,
              gc3_ref, gc4_ref, gmg0, gmg1,
              out_ref, d_scr, idx_scr):
    pcl = pcl_ref[0]
    pcl_T = pcl.T
    psq_rm = jnp.sum(pcl * pcl, axis=1, keepdims=True)
    psq_T = jnp.sum(pcl_T * pcl_T, axis=0, keepdims=True)
    d_T = _dist_T(pcl, psq_rm, pcl_T, psq_T, _NG, _NG)
    _topk_min_idx(d_T, _KG + 1, d_scr, idx_scr)

    y = _local_feat(pcl_T, [lfg_0[...], lfg_1[...], lfg_2[...], lfg_3[...]],
                    _KG, 1, idx_scr)
    y = jax.nn.relu(_mm_T(gc1_ref[...], y))
    y = jax.nn.relu(_mm_T(gc2_ref[...], y))

    y, g1 = _hier(y[:, :_MG0], gs1_0[...], gs1_1[...], _MG0)
    y, g2 = _hier(y[:, :_MG1], gs2_0[...], gs2_1[...], _MG1, g1, gs2_2[...])
    y, g3 = _hier(y[:, :_MG1], gs3_0[...], gs3_1[...], _MG1, g2, gs3_2[...])
    y = jax.nn.relu(_mm_T(gc3_ref[...], y)) + y
    y = jax.nn.relu(_mm_T(gc4_ref[...], y))
    yg = jnp.max(y, axis=1, keepdims=True)
    g = jnp.concatenate([yg, g1, g2, g3], axis=0)
    g = jax.nn.relu(_mm_T(gmg0[...], g))
    g = jax.nn.relu(_mm_T(gmg1[...], g))
    out_ref[0] = g


def _full_spec(shape):
    nd = len(shape)
    return pl.BlockSpec(shape, lambda b, _n=nd: (0,) * _n)


def kernel(pos, pcl, pre_oriented_normal, params):
    B = pos.shape[0]
    p = params

    g_weights = (p['lfg'] + [p['gc1'], p['gc2']] + p['gs1'] + p['gs2']
                 + p['gs3'] + [p['gc3'], p['gc4']] + p['gmg'])
    gg = pl.pallas_call(
        _g_kernel,
        grid=(B,),
        in_specs=[pl.BlockSpec((1, _NG, 3), lambda b: (b, 0, 0))]
                 + [_full_spec(w.shape) for w in g_weights],
        out_specs=pl.BlockSpec((1, 128, 1), lambda b: (b, 0, 0)),
        out_shape=jax.ShapeDtypeStruct((B, 128, 1), F32),
        scratch_shapes=[pltpu.VMEM((_NG, _NG), F32),
                        pltpu.VMEM((16, _NG), I32)],
        compiler_params=pltpu.CompilerParams(
            dimension_semantics=("arbitrary",)),
    )(pcl, *g_weights)

    alpha = p['alpha'].reshape(99, 1)
    pre3 = pre_oriented_normal.reshape(B, 3, 1)
    m_weights = (p['stn'] + p['lf1'] + p['lf2']
                 + [alpha, p['c1'], p['c2']]
                 + p['s1'] + p['s2'] + p['s3'] + p['s4']
                 + [p['c3'], p['c4'], p['cg']] + p['mg']
                 + [p['k1'], p['k2'], p['kw'], p['n']] + p['sg'])
    out = pl.pallas_call(
        _main_kernel,
        grid=(B,),
        in_specs=[pl.BlockSpec((1, _N, 3), lambda b: (b, 0, 0)),
                  pl.BlockSpec((1, 3, 1), lambda b: (b, 0, 0)),
                  pl.BlockSpec((1, 128, 1), lambda b: (b, 0, 0))]
                 + [_full_spec(w.shape) for w in m_weights],
        out_specs=pl.BlockSpec((1, 3, 1), lambda b: (b, 0, 0)),
        out_shape=jax.ShapeDtypeStruct((B, 3, 1), F32),
        scratch_shapes=[pltpu.VMEM((_SW, _SW), F32),
                        pltpu.VMEM((40, _SW), I32)],
        compiler_params=pltpu.CompilerParams(
            dimension_semantics=("arbitrary",)),
    )(pos, pre3, gg, *m_weights)
    return out[:, :, 0]


# speedup vs baseline: 24.0906x; 1.2914x over previous
import functools
import jax
import jax.numpy as jnp
from jax import lax
from jax.experimental import pallas as pl
from jax.experimental.pallas import tpu as pltpu

F32 = jnp.float32
I32 = jnp.int32
NEG = -jnp.inf


def _ceil128(n):
    return (n + 127) // 128 * 128


def _gather_cols(tab_T, ik_row, Q):
    C, R = tab_T.shape
    W = ik_row.shape[1]
    Rp = _ceil128(R)
    if Rp != R:
        tab_T = jnp.concatenate(
            [tab_T, jnp.zeros((C, Rp - R), dtype=F32)], axis=1)
    idxb = jnp.broadcast_to(ik_row, (C, W))
    loc = idxb % 128
    ch = idxb // 128
    chunks = [tab_T[:, j * 128:(j + 1) * 128] for j in range(Rp // 128)]
    tiles = []
    for i in range(W // 128):
        loc_i = loc[:, i * 128:(i + 1) * 128]
        ch_i = ch[:, i * 128:(i + 1) * 128]
        acc = jnp.zeros((C, 128), dtype=F32)
        for j, chunk in enumerate(chunks):
            g = jnp.take_along_axis(chunk, loc_i, axis=1,
                                    mode="promise_in_bounds")
            acc = jnp.where(ch_i == j, g, acc)
        tiles.append(acc)
    return jnp.concatenate(tiles, axis=1)[:, :Q]


def _topk_min_idx(d_T, K, d_scr, idx_scr):
    R, Q = d_T.shape
    Rp = (R + 7) // 8 * 8
    Qp = _ceil128(Q)
    d_scr[:Rp, :Qp] = jnp.full((Rp, Qp), jnp.inf, dtype=F32)
    d_scr[:R, :Q] = d_T
    iot = lax.broadcasted_iota(I32, (Rp, Qp), 0)

    def body(k, carry):
        d = d_scr[:Rp, :Qp]
        m = jnp.min(d, axis=0, keepdims=True)
        cand = jnp.where(d == m, iot, jnp.int32(2 ** 30))
        idxv = jnp.min(cand, axis=0, keepdims=True)
        idx_scr[pl.ds(k, 1), :Qp] = idxv
        d_scr[:Rp, :Qp] = jnp.where(iot == idxv, jnp.inf, d)
        return carry

    lax.fori_loop(0, K, body, 0)


def _edge_conv(x_T, W, K, k_off, idx_scr, N):
    d = x_T.shape[0]
    Wt, Wb = W[:d], W[d:]
    A = lax.dot_general(Wt - Wb, x_T, (((0,), (0,)), ((), ())),
                        preferred_element_type=F32)
    Bv = lax.dot_general(Wb, x_T, (((0,), (0,)), ((), ())),
                         preferred_element_type=F32)

    def body(k, acc):
        ik = idx_scr[pl.ds(k_off + k, 1), :_ceil128(N)]
        g = _gather_cols(Bv, ik, N)
        return jnp.maximum(acc, jax.nn.relu(A + g))

    acc0 = jnp.full((24, N), NEG, dtype=F32)
    return lax.fori_loop(0, K, body, acc0)


def _local_feat(pos_T, Ws, K, k_off, idx_scr):
    x_T = pos_T
    N = pos_T.shape[1]
    for W in Ws:
        e = _edge_conv(x_T, W, K, k_off, idx_scr, N)
        x_T = jnp.concatenate([x_T, e], axis=0)
    return x_T


def _gather_max(y_T, K, k_off, idx_scr, Q):
    C = y_T.shape[0]

    def body(k, acc):
        ik = idx_scr[pl.ds(k_off + k, 1), :_ceil128(Q)]
        g = _gather_cols(y_T, ik, Q)
        return jnp.maximum(acc, g)

    acc0 = jnp.full((C, Q), NEG, dtype=F32)
    return lax.fori_loop(0, K, body, acc0)


def _mm_T(W, x):
    return lax.dot_general(W, x, (((0,), (0,)), ((), ())),
                           preferred_element_type=F32)


def _dist_T(ref_rm, ref_sq, query_T, query_sq, R, Q):
    mm = lax.dot_general(ref_rm[:R], query_T[:, :Q],
                         (((1,), (0,)), ((), ())),
                         preferred_element_type=F32)
    return (ref_sq[:R] + query_sq[:, :Q]) - 2.0 * mm


def _qstn(pos_rm, Ws):
    h = jax.nn.relu(jnp.dot(pos_rm, Ws[0], preferred_element_type=F32))
    h = jax.nn.relu(jnp.dot(h, Ws[1], preferred_element_type=F32))
    h = jax.nn.relu(jnp.dot(h, Ws[2], preferred_element_type=F32))
    h = jnp.max(h, axis=0, keepdims=True)
    h = jax.nn.relu(jnp.dot(h, Ws[3], preferred_element_type=F32))
    h = jax.nn.relu(jnp.dot(h, Ws[4], preferred_element_type=F32))
    q = jnp.dot(h, Ws[5], preferred_element_type=F32)
    lane4 = lax.broadcasted_iota(I32, (1, 4), 1)
    q = q + jnp.where(lane4 == 0, 1.0, 0.0).astype(F32)
    q = q / (jnp.sqrt(jnp.sum(q * q, axis=1, keepdims=True)) + 1e-8)
    w, x, y, z = (q[:, 0:1], q[:, 1:2], q[:, 2:3], q[:, 3:4])
    r0 = jnp.concatenate([1 - 2 * (y * y + z * z), 2 * (x * y - w * z),
                          2 * (x * z + w * y)], axis=1)
    r1 = jnp.concatenate([2 * (x * y + w * z), 1 - 2 * (x * x + z * z),
                          2 * (y * z - w * x)], axis=1)
    r2 = jnp.concatenate([2 * (x * z - w * y), 2 * (y * z + w * x),
                          1 - 2 * (x * x + y * y)], axis=1)
    return jnp.concatenate([r0, r1, r2], axis=0)


def _hier(pooled_T, W, Wg, M, x_last=None, Wl=None):
    h = jax.nn.relu(_mm_T(W, pooled_T))
    if x_last is not None:
        h = h + jax.nn.relu(_mm_T(Wl, x_last))
    g = jnp.max(jax.nn.relu(_mm_T(Wg, h)), axis=1, keepdims=True)
    return h, g


_N = 700
_M0, _M1, _M2, _M3 = 466, 310, 206, 103
_KL1, _KL2, _KH1, _KH2 = 16, 32, 16, 32
_SW = 768


def _main_kernel(pos_ref, pre_ref, gg_ref,
                 stn0, stn1, stn2, stn3, stn4, stn5,
                 lf1_0, lf1_1, lf1_2, lf1_3,
                 lf2_0, lf2_1, lf2_2, lf2_3,
                 alpha_ref, c1_ref, c2_ref,
                 s1_0, s1_1, s2_0, s2_1, s2_2,
                 s3_0, s3_1, s3_2, s4_0, s4_1, s4_2,
                 c3_ref, c4_ref, cg_ref, mg0, mg1,
                 k1_ref, k2_ref, kw_ref, n_ref,
                 sg0, sg1, sg2,
                 out_ref, d_scr, idx_scr):
    pos = pos_ref[0]
    rot = _qstn(pos, [stn0[...], stn1[...], stn2[...],
                      stn3[...], stn4[...], stn5[...]])
    pos_rm = jnp.dot(pos, rot, preferred_element_type=F32)
    pos_T = pos_rm.T
    psq_rm = jnp.sum(pos_rm * pos_rm, axis=1, keepdims=True)
    psq_T = jnp.sum(pos_T * pos_T, axis=0, keepdims=True)

    d_T = _dist_T(pos_rm, psq_rm, pos_T, psq_T, _N, _N)
    _topk_min_idx(d_T, _KL2 + 1, d_scr, idx_scr)

    y1 = _local_feat(pos_T, [lf1_0[...], lf1_1[...], lf1_2[...], lf1_3[...]],
                     _KL1, 1, idx_scr)
    y2 = _local_feat(pos_T, [lf2_0[...], lf2_1[...], lf2_2[...], lf2_3[...]],
                     _KL2, 1, idx_scr)
    a = jax.nn.sigmoid(alpha_ref[...])
    y = a * y1 + (1.0 - a) * y2
    y = jax.nn.relu(_mm_T(c1_ref[...], y))
    y = jax.nn.relu(_mm_T(c2_ref[...], y))

    d_T = _dist_T(pos_rm, psq_rm, pos_T, psq_T, _N, _M0)
    _topk_min_idx(d_T, _KH1 + 1, d_scr, idx_scr)
    pooled = _gather_max(y, _KH1, 1, idx_scr, _M0)
    y, g1 = _hier(pooled, s1_0[...], s1_1[...], _M0)

    d_T = _dist_T(pos_rm, psq_rm, pos_T, psq_T, _M0, _M1)
    _topk_min_idx(d_T, _KH1 + 1, d_scr, idx_scr)
    pooled = _gather_max(y, _KH1, 1, idx_scr, _M1)
    y, g2 = _hier(pooled, s2_0[...], s2_1[...], _M1, g1, s2_2[...])

    d_T = _dist_T(pos_rm, psq_rm, pos_T, psq_T, _M1, _M2)
    _topk_min_idx(d_T, _KH2 + 1, d_scr, idx_scr)
    pooled = _gather_max(y, _KH2, 1, idx_scr, _M2)
    y, g3 = _hier(pooled, s3_0[...], s3_1[...], _M2, g2, s3_2[...])

    d_T = _dist_T(pos_rm, psq_rm, pos_T, psq_T, _M2, _M2)
    _topk_min_idx(d_T, _KH2 + 1, d_scr, idx_scr)
    pooled = _gather_max(y, _KH2, 1, idx_scr, _M2)
    y, g4 = _hier(pooled, s4_0[...], s4_1[...], _M2, g3, s4_2[...])

    y = jax.nn.relu(_mm_T(c3_ref[...], y)) + y
    y = jax.nn.relu(_mm_T(c4_ref[...], y))
    yg = jax.nn.relu(_mm_T(cg_ref[...], y[:, :_M3])) + y[:, :_M3]
    yg = jnp.max(yg, axis=1, keepdims=True)
    pg = jnp.concatenate([g1, g2, g3, g4, yg], axis=0)
    pg = jax.nn.relu(_mm_T(mg0[...], pg))
    pg = jax.nn.relu(_mm_T(mg1[...], pg))

    h = jax.nn.relu(_mm_T(k1_ref[...], y))
    h = jax.nn.relu(_mm_T(k2_ref[...], h))
    logit = _mm_T(kw_ref[...], h)
    logit = logit - jnp.max(logit, axis=1, keepdims=True)
    wexp = jnp.exp(logit)
    wsm = wexp / jnp.sum(wexp, axis=1, keepdims=True)
    f = jnp.sum(y * wsm, axis=1, keepdims=True)
    n = _mm_T(n_ref[...], f)
    comb = jnp.concatenate([f, pg, gg_ref[0]], axis=0)
    s = jax.nn.relu(_mm_T(sg0[...], comb))
    s = jax.nn.relu(_mm_T(sg1[...], s))
    s = _mm_T(sg2[...], s)
    nu = n / (jnp.sqrt(jnp.sum(n * n, axis=0, keepdims=True)) + 1e-8)
    flip = jnp.sign(jnp.sum(nu * pre_ref[0], axis=0, keepdims=True) + 1e-6)
    out_ref[0] = nu * jnp.tanh(s) * flip


_NG = 1024
_MG0, _MG1 = 512, 256
_KG = 8


def _g_kernel(pcl_ref,
              lfg_0, lfg_1, lfg_2, lfg_3,
              gc1_ref, gc2_ref,
              gs1_0, gs1_1, gs2_0, gs2_1, gs2_2, gs3_0, gs3_1, gs3_2,
              gc3_ref, gc4_ref, gmg0, gmg1,
              out_ref, d_scr, idx_scr):
    pcl = pcl_ref[0]
    pcl_T = pcl.T
    psq_rm = jnp.sum(pcl * pcl, axis=1, keepdims=True)
    psq_T = jnp.sum(pcl_T * pcl_T, axis=0, keepdims=True)
    d_T = _dist_T(pcl, psq_rm, pcl_T, psq_T, _NG, _NG)
    _topk_min_idx(d_T, _KG + 1, d_scr, idx_scr)

    y = _local_feat(pcl_T, [lfg_0[...], lfg_1[...], lfg_2[...], lfg_3[...]],
                    _KG, 1, idx_scr)
    y = jax.nn.relu(_mm_T(gc1_ref[...], y))
    y = jax.nn.relu(_mm_T(gc2_ref[...], y))

    y, g1 = _hier(y[:, :_MG0], gs1_0[...], gs1_1[...], _MG0)
    y, g2 = _hier(y[:, :_MG1], gs2_0[...], gs2_1[...], _MG1, g1, gs2_2[...])
    y, g3 = _hier(y[:, :_MG1], gs3_0[...], gs3_1[...], _MG1, g2, gs3_2[...])
    y = jax.nn.relu(_mm_T(gc3_ref[...], y)) + y
    y = jax.nn.relu(_mm_T(gc4_ref[...], y))
    yg = jnp.max(y, axis=1, keepdims=True)
    g = jnp.concatenate([yg, g1, g2, g3], axis=0)
    g = jax.nn.relu(_mm_T(gmg0[...], g))
    g = jax.nn.relu(_mm_T(gmg1[...], g))
    out_ref[0] = g


def _full_spec(shape):
    nd = len(shape)
    return pl.BlockSpec(shape, lambda b, _n=nd: (0,) * _n)


def kernel(pos, pcl, pre_oriented_normal, params):
    B = pos.shape[0]
    p = params

    g_weights = (p['lfg'] + [p['gc1'], p['gc2']] + p['gs1'] + p['gs2']
                 + p['gs3'] + [p['gc3'], p['gc4']] + p['gmg'])
    gg = pl.pallas_call(
        _g_kernel,
        grid=(B,),
        in_specs=[pl.BlockSpec((1, _NG, 3), lambda b: (b, 0, 0))]
                 + [_full_spec(w.shape) for w in g_weights],
        out_specs=pl.BlockSpec((1, 128, 1), lambda b: (b, 0, 0)),
        out_shape=jax.ShapeDtypeStruct((B, 128, 1), F32),
        scratch_shapes=[pltpu.VMEM((_NG, _NG), F32),
                        pltpu.VMEM((16, _NG), I32)],
        compiler_params=pltpu.CompilerParams(
            dimension_semantics=("parallel",)),
    )(pcl, *g_weights)

    alpha = p['alpha'].reshape(99, 1)
    pre3 = pre_oriented_normal.reshape(B, 3, 1)
    m_weights = (p['stn'] + p['lf1'] + p['lf2']
                 + [alpha, p['c1'], p['c2']]
                 + p['s1'] + p['s2'] + p['s3'] + p['s4']
                 + [p['c3'], p['c4'], p['cg']] + p['mg']
                 + [p['k1'], p['k2'], p['kw'], p['n']] + p['sg'])
    out = pl.pallas_call(
        _main_kernel,
        grid=(B,),
        in_specs=[pl.BlockSpec((1, _N, 3), lambda b: (b, 0, 0)),
                  pl.BlockSpec((1, 3, 1), lambda b: (b, 0, 0)),
                  pl.BlockSpec((1, 128, 1), lambda b: (b, 0, 0))]
                 + [_full_spec(w.shape) for w in m_weights],
        out_specs=pl.BlockSpec((1, 3, 1), lambda b: (b, 0, 0)),
        out_shape=jax.ShapeDtypeStruct((B, 3, 1), F32),
        scratch_shapes=[pltpu.VMEM((_SW, _SW), F32),
                        pltpu.VMEM((40, _SW), I32)],
        compiler_params=pltpu.CompilerParams(
            dimension_semantics=("parallel",)),
    )(pos, pre3, gg, *m_weights)
    return out[:, :, 0]
